# baseline (speedup 1.0000x reference)
import jax
import jax.numpy as jnp
from jax import lax
from jax.experimental import pallas as pl
from jax.experimental.pallas import tpu as pltpu
from jax.experimental.pallas import tpu_sc as plsc

N = 50000
NPAD = 51200
NPK = NPAD // 8
E = 800000
EPAD = 819200
EPK = EPAD // 8
NG = 4
RE = 1024
RN = 800
F32 = jnp.float32

NC = 2
NS = 16
NW = NC * NS
CH = 128
NCH_W = EPAD // (NW * CH)
KG = 8
NGRP = NCH_W // KG
TROWS = NPAD // NS

_SC_PARAMS = pltpu.CompilerParams(use_tc_tiling_on_sc=False)
_MESH_CACHE = []


def _mesh():
    if not _MESH_CACHE:
        _MESH_CACHE.append(
            plsc.VectorSubcoreMesh(core_axis_name="c", subcore_axis_name="s",
                                   num_cores=NC, num_subcores=NS))
    return _MESH_CACHE[0]


def _mm(a, b):
    return jnp.dot(a, b, preferred_element_type=F32)


def _enc_body(bc_ref, m_ref, c_ref, w1_ref, b1_ref, w2_ref, b2_ref,
              w3_ref, b3_ref, out_ref):
    enc = _mm(bc_ref[...], m_ref[...]) + c_ref[...]
    f = jnp.maximum(_mm(enc, w1_ref[...]) + b1_ref[...], 0.0)
    f = jnp.maximum(_mm(f, w2_ref[...]) + b2_ref[...], 0.0)
    out_ref[...] = _mm(f, w3_ref[...]) + b3_ref[...]


def _n0_body(xs_ref, bat_ref, fc4_ref, out_ref):
    xs = xs_ref[...]
    bat = bat_ref[...]
    acc = jnp.zeros(xs.shape, F32)
    zero8 = jnp.zeros((1, 8), F32)
    for g in range(NG):
        row16 = jnp.concatenate([zero8, fc4_ref[g:g + 1, :]], axis=1)
        row = jnp.concatenate([row16] * 8, axis=1)
        acc = acc + jnp.where(bat == g, row, 0.0)
    out_ref[...] = xs + acc


def _emlp0_body(gs_ref, gd_ref, ea_ref, w1s_ref, w1d_ref, w1e_ref, b1_ref,
                w2_ref, b2_ref, out_ref):
    gs = gs_ref[...]
    gd = gd_ref[...]
    ea = ea_ref[...]
    for k in range(8):
        o = 16 * k
        m = _mm(gs[:, o:o + 16], w1s_ref[...])
        m = m + _mm(gd[:, o:o + 16], w1d_ref[...])
        m = m + _mm(ea[:, o:o + 16], w1e_ref[...])
        m = jnp.maximum(m + b1_ref[...], 0.0)
        out_ref[:, o:o + 16] = jnp.maximum(_mm(m, w2_ref[...]) + b2_ref[...],
                                           0.0)


def _emlp1_body(g1s_ref, g1d_ref, g0s_ref, g0d_ref, e0_ref, wa_ref, wb_ref,
                wc_ref, wd_ref, we_ref, b1_ref, w2_ref, b2_ref, out_ref):
    g1s = g1s_ref[...]
    g1d = g1d_ref[...]
    g0s = g0s_ref[...]
    g0d = g0d_ref[...]
    e0 = e0_ref[...]
    for k in range(8):
        o = 16 * k
        a = g1s[:, o:o + 16]
        d = g1d[:, o:o + 16]
        es = (e0[:, o:o + 16] + a + d) * (1.0 / 3.0)
        m = _mm(a, wa_ref[...])
        m = m + _mm(g0s[:, o + 6:o + 8], wb_ref[...])
        m = m + _mm(d, wc_ref[...])
        m = m + _mm(g0d[:, o + 6:o + 8], wd_ref[...])
        m = m + _mm(es, we_ref[...])
        m = jnp.maximum(m + b1_ref[...], 0.0)
        out_ref[:, o:o + 16] = jnp.maximum(_mm(m, w2_ref[...]) + b2_ref[...],
                                           0.0)


def _nmlp0_body(n_ref, p0_ref, p1_ref, wa_ref, wb_ref, b1_ref, w2_ref,
                b2_ref, out_ref):
    n = n_ref[...]
    agg = p0_ref[...] + p1_ref[...]
    for k in range(8):
        o = 16 * k
        h = _mm(n[:, o:o + 16], wa_ref[...]) + _mm(agg[:, o:o + 16],
                                                   wb_ref[...])
        h = jnp.maximum(h + b1_ref[...], 0.0)
        out_ref[:, o:o + 16] = jnp.maximum(_mm(h, w2_ref[...]) + b2_ref[...],
                                           0.0)


def _nmlp1_body(xs_ref, n0_ref, p0_ref, p1_ref, wa_ref, wb_ref, wc_ref,
                b1_ref, w2_ref, b2_ref, out_ref):
    xs = xs_ref[...]
    n0 = n0_ref[...]
    agg = p0_ref[...] + p1_ref[...]
    for k in range(8):
        o = 16 * k
        h = _mm(xs[:, o:o + 16], wa_ref[...])
        h = h + _mm(n0[:, o + 6:o + 8], wb_ref[...])
        h = h + _mm(agg[:, o:o + 16], wc_ref[...])
        h = jnp.maximum(h + b1_ref[...], 0.0)
        out_ref[:, o:o + 16] = jnp.maximum(_mm(h, w2_ref[...]) + b2_ref[...],
                                           0.0)


def _smooth_body(xn_ref, nb0_ref, nb1_ref, dg0_ref, dg1_ref, out_ref):
    xn = xn_ref[...]
    nbr = nb0_ref[...] + nb1_ref[...]
    dg = dg0_ref[...] + dg1_ref[...]
    for k in range(8):
        o = 16 * k
        deg = dg[:, o:o + 1]
        out_ref[:, o:o + 16] = ((xn[:, o:o + 16] + nbr[:, o:o + 16])
                                / (1.0 + deg))


def _smooth_dec_body(xn_ref, nb0_ref, nb1_ref, dg0_ref, dg1_ref, dw_ref,
                     db_ref, out_ref):
    xn = xn_ref[...]
    nbr = nb0_ref[...] + nb1_ref[...]
    dg = dg0_ref[...] + dg1_ref[...]
    for k in range(8):
        o = 16 * k
        deg = dg[:, o:o + 1]
        xs = (xn[:, o:o + 16] + nbr[:, o:o + 16]) / (1.0 + deg)
        out_ref[:, 4 * k:4 * k + 4] = _mm(xs, dw_ref[...]) + db_ref[...]


def _row_spec(b, w):
    return pl.BlockSpec((b, w), lambda i: (i, 0))


def _full_spec(shape):
    return pl.BlockSpec(shape, lambda i: tuple(0 for _ in shape))


def _rows_call(body, nrows, block, in_arrays, widths, out_w):
    grid = nrows // block
    specs = []
    for a, w in zip(in_arrays, widths):
        specs.append(_row_spec(block, w) if w is not None else _full_spec(a.shape))
    return pl.pallas_call(
        body,
        grid=(grid,),
        in_specs=specs,
        out_specs=_row_spec(block, out_w),
        out_shape=jax.ShapeDtypeStruct((nrows, out_w), F32),
    )(*in_arrays)


def _worker_chunk_base():
    wid = lax.axis_index("s") * NC + lax.axis_index("c")
    return wid * NCH_W


def _fill_rows(buf, vec):
    def body(r, _):
        buf[r] = vec
        return 0
    lax.fori_loop(0, CH, body, 0)


def _zero_acc_slice(zbuf, acc):
    t = lax.axis_index("s")
    def body(j, _):
        pltpu.sync_copy(zbuf, acc.at[pl.ds(t * TROWS + j * CH, CH)])
        return 0
    lax.fori_loop(0, TROWS // CH, body, 0)


def _copy_out(acc, p0, p1, obuf):
    c = lax.axis_index("c")
    t = lax.axis_index("s")
    def body(j, _):
        r0 = t * TROWS + j * CH
        pltpu.sync_copy(acc.at[pl.ds(r0, CH)], obuf)
        @pl.when(c == 0)
        def _():
            pltpu.sync_copy(obuf, p0.at[pl.ds(r0, CH)])
        @pl.when(c == 1)
        def _():
            pltpu.sync_copy(obuf, p1.at[pl.ds(r0, CH)])
        return 0
    lax.fori_loop(0, TROWS // CH, body, 0)


def _gather_kernel_body(tab, idx, out, idx_v, rbufs, gsem, wsem):
    base = _worker_chunk_base()
    pltpu.sync_copy(idx.at[pl.ds(base, NCH_W)], idx_v)

    def issue(g, s):
        for b in range(KG):
            pltpu.async_copy(tab.at[idx_v.at[g * KG + b]],
                             rbufs.at[s].at[b], gsem)

    issue(0, 0)

    def body(i, _):
        s = lax.rem(i, 2)
        for b in range(KG):
            pltpu.make_async_copy(tab.at[idx_v.at[i * KG + b]],
                                  rbufs.at[s].at[b], gsem).wait()
        @pl.when(i + 1 < NGRP)
        def _():
            issue(i + 1, 1 - s)
        for b in range(KG):
            pltpu.async_copy(rbufs.at[s].at[b],
                             out.at[pl.ds((base + i * KG + b) * CH, CH)],
                             wsem)
        for b in range(KG):
            pltpu.make_async_copy(rbufs.at[s].at[b],
                                  out.at[pl.ds((base + i * KG + b) * CH, CH)],
                                  wsem).wait()
        return 0

    lax.fori_loop(0, NGRP, body, 0)


def _gather_rows(table, idx2d):
    out = pl.kernel(
        _gather_kernel_body,
        out_type=jax.ShapeDtypeStruct((EPAD, 16), F32),
        mesh=_mesh(),
        compiler_params=_SC_PARAMS,
        scratch_types=[
            pltpu.VMEM((NCH_W, CH), jnp.int32),
            pltpu.VMEM((2, KG, CH, 16), F32),
            pltpu.SemaphoreType.DMA,
            pltpu.SemaphoreType.DMA,
        ],
    )(table.reshape(NPAD, 16), idx2d)
    return out.reshape(EPK, 128)


def _scatter_kernel_body(vals, idx, p0, p1, idx_v, vbufs, sbuf, zbuf,
                         obuf, acc, lsem):
    base = _worker_chunk_base()
    pltpu.sync_copy(idx.at[pl.ds(base, NCH_W)], idx_v)
    _fill_rows(zbuf, jnp.zeros((16,), F32))
    _zero_acc_slice(zbuf, acc)
    plsc.subcore_barrier()

    PR = CH // 8

    def issue(g, s):
        for b in range(KG):
            pltpu.async_copy(vals.at[pl.ds((base + g * KG + b) * PR, PR)],
                             vbufs.at[s].at[b], lsem)

    issue(0, 0)

    def body(i, _):
        s = lax.rem(i, 2)
        for b in range(KG):
            pltpu.make_async_copy(vals.at[pl.ds((base + i * KG + b) * PR, PR)],
                                  vbufs.at[s].at[b], lsem).wait()
        @pl.when(i + 1 < NGRP)
        def _():
            issue(i + 1, 1 - s)
        for b in range(KG):
            pkd = vbufs.at[s].at[b]
            def tr(r, _):
                for k in range(8):
                    sbuf[8 * r + k] = pkd[r, 16 * k:16 * k + 16]
                return 0
            lax.fori_loop(0, PR, tr, 0)
            pltpu.sync_copy(sbuf, acc.at[idx_v.at[i * KG + b]], add=True)
        return 0

    lax.fori_loop(0, NGRP, body, 0)
    plsc.subcore_barrier()
    _copy_out(acc, p0, p1, obuf)


def _scatter_partials(vals, dst2d):
    sds = jax.ShapeDtypeStruct((NPAD, 16), F32)
    p0, p1 = pl.kernel(
        _scatter_kernel_body,
        out_type=(sds, sds),
        mesh=_mesh(),
        compiler_params=_SC_PARAMS,
        scratch_types=[
            pltpu.VMEM((NCH_W, CH), jnp.int32),
            pltpu.VMEM((2, KG, CH // 8, 128), F32),
            pltpu.VMEM((CH, 16), F32),
            pltpu.VMEM((CH, 16), F32),
            pltpu.VMEM((CH, 16), F32),
            pltpu.VMEM_SHARED((NPAD, 16), F32),
            pltpu.SemaphoreType.DMA,
        ],
    )(vals, dst2d)
    return p0.reshape(NPK, 128), p1.reshape(NPK, 128)


def _deg_kernel_body(idx, d0, d1, idx_v, ones_v, zbuf, obuf, acc):
    base = _worker_chunk_base()
    pltpu.sync_copy(idx.at[pl.ds(base, NCH_W)], idx_v)
    _fill_rows(zbuf, jnp.zeros((16,), F32))
    _fill_rows(ones_v, jnp.ones((16,), F32))
    _zero_acc_slice(zbuf, acc)
    plsc.subcore_barrier()

    def body(i, _):
        pltpu.sync_copy(ones_v, acc.at[idx_v.at[i]], add=True)
        return 0

    lax.fori_loop(0, NCH_W, body, 0)
    plsc.subcore_barrier()
    _copy_out(acc, d0, d1, obuf)


def _deg_partials(dst2d):
    sds = jax.ShapeDtypeStruct((NPAD, 16), F32)
    d0, d1 = pl.kernel(
        _deg_kernel_body,
        out_type=(sds, sds),
        mesh=_mesh(),
        compiler_params=_SC_PARAMS,
        scratch_types=[
            pltpu.VMEM((NCH_W, CH), jnp.int32),
            pltpu.VMEM((CH, 16), F32),
            pltpu.VMEM((CH, 16), F32),
            pltpu.VMEM((CH, 16), F32),
            pltpu.VMEM_SHARED((NPAD, 16), F32),
        ],
    )(dst2d)
    return d0.reshape(NPK, 128), d1.reshape(NPK, 128)


def _gs_kernel_body(tab, idx_s, idx_d, p0, p1, sidx_v, didx_v, vbufs, zbuf,
                    obuf, acc, lsem):
    KF = KG // 2
    base = _worker_chunk_base()
    pltpu.sync_copy(idx_s.at[pl.ds(base, NCH_W)], sidx_v)
    pltpu.sync_copy(idx_d.at[pl.ds(base, NCH_W)], didx_v)
    _fill_rows(zbuf, jnp.zeros((16,), F32))
    _zero_acc_slice(zbuf, acc)
    plsc.subcore_barrier()

    def issue(g, s):
        for b in range(KF):
            pltpu.async_copy(tab.at[sidx_v.at[g * KF + b]],
                             vbufs.at[s].at[b], lsem)

    issue(0, 0)
    ngrp = NCH_W // KF

    def body(i, _):
        s = lax.rem(i, 2)
        for b in range(KF):
            pltpu.make_async_copy(tab.at[sidx_v.at[i * KF + b]],
                                  vbufs.at[s].at[b], lsem).wait()
        @pl.when(i + 1 < ngrp)
        def _():
            issue(i + 1, 1 - s)
        for b in range(KF):
            pltpu.sync_copy(vbufs.at[s].at[b],
                            acc.at[didx_v.at[i * KF + b]], add=True)
        return 0

    lax.fori_loop(0, ngrp, body, 0)
    plsc.subcore_barrier()
    _copy_out(acc, p0, p1, obuf)


def _gather_scatter(table, sidx2d, didx2d):
    KF = KG // 2
    sds = jax.ShapeDtypeStruct((NPAD, 16), F32)
    p0, p1 = pl.kernel(
        _gs_kernel_body,
        out_type=(sds, sds),
        mesh=_mesh(),
        compiler_params=_SC_PARAMS,
        scratch_types=[
            pltpu.VMEM((NCH_W, CH), jnp.int32),
            pltpu.VMEM((NCH_W, CH), jnp.int32),
            pltpu.VMEM((2, KF, CH, 16), F32),
            pltpu.VMEM((CH, 16), F32),
            pltpu.VMEM((CH, 16), F32),
            pltpu.VMEM_SHARED((NPAD, 16), F32),
            pltpu.SemaphoreType.DMA,
        ],
    )(table.reshape(NPAD, 16), sidx2d, didx2d)
    return p0.reshape(NPK, 128), p1.reshape(NPK, 128)


def kernel(x, edge_index, edge_attr, bc, batch, params):
    p = params
    src = edge_index[0]
    dst = edge_index[1]
    epad = EPAD - E
    src_p = jnp.concatenate([src, jnp.zeros((epad,), jnp.int32)]
                            ).reshape(EPAD // CH, CH)
    dst_g = jnp.concatenate([dst, jnp.zeros((epad,), jnp.int32)]
                            ).reshape(EPAD // CH, CH)
    dst_s = jnp.concatenate([dst, jnp.full((epad,), N, jnp.int32)]
                            ).reshape(EPAD // CH, CH)
    ea16 = jnp.pad(edge_attr, ((0, epad), (0, 12))).reshape(EPK, 128)

    npad = NPAD - N
    xs16 = jnp.pad(jnp.concatenate([x, x[:, :2]], axis=1),
                   ((0, npad), (0, 8))).reshape(NPK, 128)
    bat_p = jnp.broadcast_to(
        jnp.pad(batch, (0, npad), constant_values=NG - 1
                ).reshape(NPK, 8)[:, :, None],
        (NPK, 8, 16)).reshape(NPK, 128)

    def conv_stack(b):
        h = lax.conv_general_dilated(b, p['w_c1'], (1,), 'VALID',
                                     dimension_numbers=('NCH', 'OIH', 'NCH'))
        h = h + p['b_c1'][None, :, None]
        h = lax.conv_general_dilated(h, p['w_c2'], (2,), 'VALID',
                                     dimension_numbers=('NCH', 'OIH', 'NCH'))
        h = h + p['b_c2'][None, :, None]
        h = lax.conv_general_dilated(h, p['w_c3'], (2,), 'VALID',
                                     dimension_numbers=('NCH', 'OIH', 'NCH'))
        h = h + p['b_c3'][None, :, None]
        h = lax.conv_general_dilated(h, p['w_c4'], (1,), 'VALID',
                                     dimension_numbers=('NCH', 'OIH', 'NCH'))
        h = h + p['b_c4'][None, :, None]
        return jnp.squeeze(h, axis=1)

    basis = jnp.concatenate([jnp.zeros((1, 1, 128), F32),
                             jnp.eye(128, dtype=F32)[:, None, :]], axis=0)
    eb = conv_stack(basis)
    c_eff = eb[0]
    m_eff = eb[1:] - c_eff[None, :]

    bc2 = bc[:, 0, :]

    fc4 = pl.pallas_call(
        _enc_body,
        out_shape=jax.ShapeDtypeStruct((NG, 8), F32),
    )(bc2, m_eff, c_eff[None, :], p['fc_w1'], p['fc_b1'][None, :],
      p['fc_w2'], p['fc_b2'][None, :], p['fc_w3'], p['fc_b3'][None, :])

    n0 = _rows_call(_n0_body, NPK, RN, (xs16, bat_p, fc4),
                    (128, 128, None), 128)

    g0s = _gather_rows(n0, src_p)
    g0d = _gather_rows(n0, dst_g)
    w1 = p['pe0_w1']
    w1e = jnp.pad(w1[32:36], ((0, 12), (0, 0)))
    e0 = _rows_call(
        _emlp0_body, EPK, RE,
        (g0s, g0d, ea16, w1[0:16], w1[16:32], w1e,
         p['pe0_b1'][None, :], p['pe0_w2'], p['pe0_b2'][None, :]),
        (128, 128, 128, None, None, None, None, None, None), 128)
    p0a, p1a = _scatter_partials(e0, dst_s)
    wn = p['pn0_w1']
    xn0 = _rows_call(
        _nmlp0_body, NPK, RN,
        (n0, p0a, p1a, wn[0:16], wn[16:32], p['pn0_b1'][None, :],
         p['pn0_w2'], p['pn0_b2'][None, :]),
        (128, 128, 128, None, None, None, None, None), 128)

    dg0, dg1 = _deg_partials(dst_s)
    nb0, nb1 = _gather_scatter(xn0, src_p, dst_s)
    xs0 = _rows_call(_smooth_body, NPK, RN, (xn0, nb0, nb1, dg0, dg1),
                     (128, 128, 128, 128, 128), 128)

    g1s = _gather_rows(xs0, src_p)
    g1d = _gather_rows(xs0, dst_g)
    w1 = p['pe1_w1']
    e1 = _rows_call(
        _emlp1_body, EPK, RE,
        (g1s, g1d, g0s, g0d, e0, w1[0:16], w1[16:18], w1[18:34], w1[34:36],
         w1[36:52], p['pe1_b1'][None, :], p['pe1_w2'], p['pe1_b2'][None, :]),
        (128, 128, 128, 128, 128, None, None, None, None, None, None, None,
         None), 128)
    p0b, p1b = _scatter_partials(e1, dst_s)
    wn = p['pn1_w1']
    xn1 = _rows_call(
        _nmlp1_body, NPK, RN,
        (xs0, n0, p0b, p1b, wn[0:16], wn[16:18], wn[18:34],
         p['pn1_b1'][None, :], p['pn1_w2'], p['pn1_b2'][None, :]),
        (128, 128, 128, 128, None, None, None, None, None, None), 128)

    nb0, nb1 = _gather_scatter(xn1, src_p, dst_s)
    pred_p = _rows_call(_smooth_dec_body, NPK, RN,
                        (xn1, nb0, nb1, dg0, dg1, p['dec_w'],
                         p['dec_b'][None, :]),
                        (128, 128, 128, 128, 128, None, None), 32)
    return pred_p.reshape(NPAD, 4)[:N]


# speedup vs baseline: 1.0163x; 1.0163x over previous
import jax
import jax.numpy as jnp
from jax import lax
from jax.experimental import pallas as pl
from jax.experimental.pallas import tpu as pltpu
from jax.experimental.pallas import tpu_sc as plsc

N = 50000
NPAD = 51200
NPK = NPAD // 8
E = 800000
EPAD = 819200
EPK = EPAD // 8
NG = 4
RE = 1024
RN = 800
F32 = jnp.float32

NC = 2
NS = 16
NW = NC * NS
CH = 128
NCH_W = EPAD // (NW * CH)
KG = 8
NGRP = NCH_W // KG
TROWS = NPAD // NS

_SC_PARAMS = pltpu.CompilerParams(use_tc_tiling_on_sc=False)
_MESH_CACHE = []


def _mesh():
    if not _MESH_CACHE:
        _MESH_CACHE.append(
            plsc.VectorSubcoreMesh(core_axis_name="c", subcore_axis_name="s",
                                   num_cores=NC, num_subcores=NS))
    return _MESH_CACHE[0]


def _mm(a, b):
    return jnp.dot(a, b, preferred_element_type=F32)


def _enc_body(bc_ref, m_ref, c_ref, w1_ref, b1_ref, w2_ref, b2_ref,
              w3_ref, b3_ref, out_ref):
    enc = _mm(bc_ref[...], m_ref[...]) + c_ref[...]
    f = jnp.maximum(_mm(enc, w1_ref[...]) + b1_ref[...], 0.0)
    f = jnp.maximum(_mm(f, w2_ref[...]) + b2_ref[...], 0.0)
    out_ref[...] = _mm(f, w3_ref[...]) + b3_ref[...]


def _n0_body(xs_ref, bat_ref, fc4_ref, out_ref):
    xs = xs_ref[...]
    bat = bat_ref[...]
    acc = jnp.zeros(xs.shape, F32)
    zero8 = jnp.zeros((1, 8), F32)
    for g in range(NG):
        row16 = jnp.concatenate([zero8, fc4_ref[g:g + 1, :]], axis=1)
        row = jnp.concatenate([row16] * 8, axis=1)
        acc = acc + jnp.where(bat == g, row, 0.0)
    out_ref[...] = xs + acc


def _emlp0_body(gs_ref, gd_ref, ea_ref, w1s_ref, w1d_ref, w1e_ref, b1_ref,
                w2_ref, b2_ref, out_ref):
    gs = gs_ref[...]
    gd = gd_ref[...]
    ea = ea_ref[...]
    for k in range(8):
        o = 16 * k
        m = _mm(gs[:, o:o + 16], w1s_ref[...])
        m = m + _mm(gd[:, o:o + 16], w1d_ref[...])
        m = m + _mm(ea[:, o:o + 16], w1e_ref[...])
        m = jnp.maximum(m + b1_ref[...], 0.0)
        out_ref[:, o:o + 16] = jnp.maximum(_mm(m, w2_ref[...]) + b2_ref[...],
                                           0.0)


def _emlp1_body(g1s_ref, g1d_ref, g0s_ref, g0d_ref, e0_ref, wa_ref, wb_ref,
                wc_ref, wd_ref, we_ref, b1_ref, w2_ref, b2_ref, out_ref):
    g1s = g1s_ref[...]
    g1d = g1d_ref[...]
    g0s = g0s_ref[...]
    g0d = g0d_ref[...]
    e0 = e0_ref[...]
    for k in range(8):
        o = 16 * k
        a = g1s[:, o:o + 16]
        d = g1d[:, o:o + 16]
        es = (e0[:, o:o + 16] + a + d) * (1.0 / 3.0)
        m = _mm(a, wa_ref[...])
        m = m + _mm(g0s[:, o + 6:o + 8], wb_ref[...])
        m = m + _mm(d, wc_ref[...])
        m = m + _mm(g0d[:, o + 6:o + 8], wd_ref[...])
        m = m + _mm(es, we_ref[...])
        m = jnp.maximum(m + b1_ref[...], 0.0)
        out_ref[:, o:o + 16] = jnp.maximum(_mm(m, w2_ref[...]) + b2_ref[...],
                                           0.0)


def _nmlp0_body(n_ref, p0_ref, p1_ref, wa_ref, wb_ref, b1_ref, w2_ref,
                b2_ref, out_ref):
    n = n_ref[...]
    agg = p0_ref[...] + p1_ref[...]
    for k in range(8):
        o = 16 * k
        h = _mm(n[:, o:o + 16], wa_ref[...]) + _mm(agg[:, o:o + 16],
                                                   wb_ref[...])
        h = jnp.maximum(h + b1_ref[...], 0.0)
        out_ref[:, o:o + 16] = jnp.maximum(_mm(h, w2_ref[...]) + b2_ref[...],
                                           0.0)


def _nmlp1_body(xs_ref, n0_ref, p0_ref, p1_ref, wa_ref, wb_ref, wc_ref,
                b1_ref, w2_ref, b2_ref, out_ref):
    xs = xs_ref[...]
    n0 = n0_ref[...]
    agg = p0_ref[...] + p1_ref[...]
    for k in range(8):
        o = 16 * k
        h = _mm(xs[:, o:o + 16], wa_ref[...])
        h = h + _mm(n0[:, o + 6:o + 8], wb_ref[...])
        h = h + _mm(agg[:, o:o + 16], wc_ref[...])
        h = jnp.maximum(h + b1_ref[...], 0.0)
        out_ref[:, o:o + 16] = jnp.maximum(_mm(h, w2_ref[...]) + b2_ref[...],
                                           0.0)


def _smooth_body(xn_ref, nb0_ref, nb1_ref, dg0_ref, dg1_ref, out_ref):
    xn = xn_ref[...]
    nbr = nb0_ref[...] + nb1_ref[...]
    dg = dg0_ref[...] + dg1_ref[...]
    for k in range(8):
        o = 16 * k
        deg = dg[:, o:o + 1]
        out_ref[:, o:o + 16] = ((xn[:, o:o + 16] + nbr[:, o:o + 16])
                                / (1.0 + deg))


def _smooth_dec_body(xn_ref, nb0_ref, nb1_ref, dg0_ref, dg1_ref, dw_ref,
                     db_ref, out_ref):
    xn = xn_ref[...]
    nbr = nb0_ref[...] + nb1_ref[...]
    dg = dg0_ref[...] + dg1_ref[...]
    for k in range(8):
        o = 16 * k
        deg = dg[:, o:o + 1]
        xs = (xn[:, o:o + 16] + nbr[:, o:o + 16]) / (1.0 + deg)
        out_ref[:, 4 * k:4 * k + 4] = _mm(xs, dw_ref[...]) + db_ref[...]


def _row_spec(b, w):
    return pl.BlockSpec((b, w), lambda i: (i, 0))


def _full_spec(shape):
    return pl.BlockSpec(shape, lambda i: tuple(0 for _ in shape))


def _rows_call(body, nrows, block, in_arrays, widths, out_w):
    grid = nrows // block
    specs = []
    for a, w in zip(in_arrays, widths):
        specs.append(_row_spec(block, w) if w is not None else _full_spec(a.shape))
    return pl.pallas_call(
        body,
        grid=(grid,),
        in_specs=specs,
        out_specs=_row_spec(block, out_w),
        out_shape=jax.ShapeDtypeStruct((nrows, out_w), F32),
    )(*in_arrays)


def _worker_chunk_base():
    wid = lax.axis_index("s") * NC + lax.axis_index("c")
    return wid * NCH_W


def _fill_rows(buf, vec):
    def body(r, _):
        buf[r] = vec
        return 0
    lax.fori_loop(0, CH, body, 0)


def _zero_acc_slice(zbuf, acc):
    t = lax.axis_index("s")
    def body(j, _):
        pltpu.sync_copy(zbuf, acc.at[pl.ds(t * TROWS + j * CH, CH)])
        return 0
    lax.fori_loop(0, TROWS // CH, body, 0)


def _copy_out(acc, p0, p1, obuf):
    c = lax.axis_index("c")
    t = lax.axis_index("s")
    def body(j, _):
        r0 = t * TROWS + j * CH
        pltpu.sync_copy(acc.at[pl.ds(r0, CH)], obuf)
        @pl.when(c == 0)
        def _():
            pltpu.sync_copy(obuf, p0.at[pl.ds(r0, CH)])
        @pl.when(c == 1)
        def _():
            pltpu.sync_copy(obuf, p1.at[pl.ds(r0, CH)])
        return 0
    lax.fori_loop(0, TROWS // CH, body, 0)


def _gather_kernel_body(tab, idx, out, idx_v, rbufs, gsem, wsem):
    base = _worker_chunk_base()
    pltpu.sync_copy(idx.at[pl.ds(base, NCH_W)], idx_v)

    def issue(g, s):
        for b in range(KG):
            pltpu.async_copy(tab.at[idx_v.at[g * KG + b]],
                             rbufs.at[s].at[b], gsem)

    issue(0, 0)

    def body(i, _):
        s = lax.rem(i, 2)
        for b in range(KG):
            pltpu.make_async_copy(tab.at[idx_v.at[i * KG + b]],
                                  rbufs.at[s].at[b], gsem).wait()
        @pl.when(i + 1 < NGRP)
        def _():
            issue(i + 1, 1 - s)
        for b in range(KG):
            pltpu.async_copy(rbufs.at[s].at[b],
                             out.at[pl.ds((base + i * KG + b) * CH, CH)],
                             wsem)
        for b in range(KG):
            pltpu.make_async_copy(rbufs.at[s].at[b],
                                  out.at[pl.ds((base + i * KG + b) * CH, CH)],
                                  wsem).wait()
        return 0

    lax.fori_loop(0, NGRP, body, 0)


def _gather_rows(table, idx2d):
    out = pl.kernel(
        _gather_kernel_body,
        out_type=jax.ShapeDtypeStruct((EPAD, 16), F32),
        mesh=_mesh(),
        compiler_params=_SC_PARAMS,
        scratch_types=[
            pltpu.VMEM((NCH_W, CH), jnp.int32),
            pltpu.VMEM((2, KG, CH, 16), F32),
            pltpu.SemaphoreType.DMA,
            pltpu.SemaphoreType.DMA,
        ],
    )(table.reshape(NPAD, 16), idx2d)
    return out.reshape(EPK, 128)


def _scatter_kernel_body(vals, idx, p0, p1, idx_v, vbufs, sbuf, zbuf,
                         obuf, acc, lsem):
    base = _worker_chunk_base()
    pltpu.sync_copy(idx.at[pl.ds(base, NCH_W)], idx_v)
    _fill_rows(zbuf, jnp.zeros((16,), F32))
    _zero_acc_slice(zbuf, acc)
    plsc.subcore_barrier()

    PR = CH // 8

    def issue(g, s):
        for b in range(KG):
            pltpu.async_copy(vals.at[pl.ds((base + g * KG + b) * PR, PR)],
                             vbufs.at[s].at[b], lsem)

    issue(0, 0)

    def body(i, _):
        s = lax.rem(i, 2)
        for b in range(KG):
            pltpu.make_async_copy(vals.at[pl.ds((base + i * KG + b) * PR, PR)],
                                  vbufs.at[s].at[b], lsem).wait()
        @pl.when(i + 1 < NGRP)
        def _():
            issue(i + 1, 1 - s)
        for b in range(KG):
            pkd = vbufs.at[s].at[b]
            def tr(r, _):
                for k in range(8):
                    sbuf[8 * r + k] = pkd[r, 16 * k:16 * k + 16]
                return 0
            lax.fori_loop(0, PR, tr, 0)
            pltpu.sync_copy(sbuf, acc.at[idx_v.at[i * KG + b]], add=True)
        return 0

    lax.fori_loop(0, NGRP, body, 0)
    plsc.subcore_barrier()
    _copy_out(acc, p0, p1, obuf)


def _scatter_partials(vals, dst2d):
    sds = jax.ShapeDtypeStruct((NPAD, 16), F32)
    p0, p1 = pl.kernel(
        _scatter_kernel_body,
        out_type=(sds, sds),
        mesh=_mesh(),
        compiler_params=_SC_PARAMS,
        scratch_types=[
            pltpu.VMEM((NCH_W, CH), jnp.int32),
            pltpu.VMEM((2, KG, CH // 8, 128), F32),
            pltpu.VMEM((CH, 16), F32),
            pltpu.VMEM((CH, 16), F32),
            pltpu.VMEM((CH, 16), F32),
            pltpu.VMEM_SHARED((NPAD, 16), F32),
            pltpu.SemaphoreType.DMA,
        ],
    )(vals, dst2d)
    return p0.reshape(NPK, 128), p1.reshape(NPK, 128)


def _deg_kernel_body(idx, d0, d1, idx_v, ones_v, zbuf, obuf, acc):
    base = _worker_chunk_base()
    pltpu.sync_copy(idx.at[pl.ds(base, NCH_W)], idx_v)
    _fill_rows(zbuf, jnp.zeros((16,), F32))
    _fill_rows(ones_v, jnp.ones((16,), F32))
    _zero_acc_slice(zbuf, acc)
    plsc.subcore_barrier()

    def body(i, _):
        pltpu.sync_copy(ones_v, acc.at[idx_v.at[i]], add=True)
        return 0

    lax.fori_loop(0, NCH_W, body, 0)
    plsc.subcore_barrier()
    _copy_out(acc, d0, d1, obuf)


def _deg_partials(dst2d):
    sds = jax.ShapeDtypeStruct((NPAD, 16), F32)
    d0, d1 = pl.kernel(
        _deg_kernel_body,
        out_type=(sds, sds),
        mesh=_mesh(),
        compiler_params=_SC_PARAMS,
        scratch_types=[
            pltpu.VMEM((NCH_W, CH), jnp.int32),
            pltpu.VMEM((CH, 16), F32),
            pltpu.VMEM((CH, 16), F32),
            pltpu.VMEM((CH, 16), F32),
            pltpu.VMEM_SHARED((NPAD, 16), F32),
        ],
    )(dst2d)
    return d0.reshape(NPK, 128), d1.reshape(NPK, 128)


def _gs_kernel_body(tab, idx_s, idx_d, p0, p1, sidx_v, didx_v, vbufs, zbuf,
                    obuf, acc, lsem):
    KF = KG // 2
    base = _worker_chunk_base()
    pltpu.sync_copy(idx_s.at[pl.ds(base, NCH_W)], sidx_v)
    pltpu.sync_copy(idx_d.at[pl.ds(base, NCH_W)], didx_v)
    _fill_rows(zbuf, jnp.zeros((16,), F32))
    _zero_acc_slice(zbuf, acc)
    plsc.subcore_barrier()

    def issue(g, s):
        for b in range(KF):
            pltpu.async_copy(tab.at[sidx_v.at[g * KF + b]],
                             vbufs.at[s].at[b], lsem)

    issue(0, 0)
    ngrp = NCH_W // KF

    def body(i, _):
        s = lax.rem(i, 2)
        for b in range(KF):
            pltpu.make_async_copy(tab.at[sidx_v.at[i * KF + b]],
                                  vbufs.at[s].at[b], lsem).wait()
        @pl.when(i + 1 < ngrp)
        def _():
            issue(i + 1, 1 - s)
        for b in range(KF):
            pltpu.sync_copy(vbufs.at[s].at[b],
                            acc.at[didx_v.at[i * KF + b]], add=True)
        return 0

    lax.fori_loop(0, ngrp, body, 0)
    plsc.subcore_barrier()
    _copy_out(acc, p0, p1, obuf)


def _gather_scatter(table, sidx2d, didx2d):
    KF = KG // 2
    sds = jax.ShapeDtypeStruct((NPAD, 16), F32)
    p0, p1 = pl.kernel(
        _gs_kernel_body,
        out_type=(sds, sds),
        mesh=_mesh(),
        compiler_params=_SC_PARAMS,
        scratch_types=[
            pltpu.VMEM((NCH_W, CH), jnp.int32),
            pltpu.VMEM((NCH_W, CH), jnp.int32),
            pltpu.VMEM((2, KF, CH, 16), F32),
            pltpu.VMEM((CH, 16), F32),
            pltpu.VMEM((CH, 16), F32),
            pltpu.VMEM_SHARED((NPAD, 16), F32),
            pltpu.SemaphoreType.DMA,
        ],
    )(table.reshape(NPAD, 16), sidx2d, didx2d)
    return p0.reshape(NPK, 128), p1.reshape(NPK, 128)


def kernel(x, edge_index, edge_attr, bc, batch, params):
    p = params
    src = edge_index[0]
    dst = edge_index[1]
    epad = EPAD - E
    src_p = jnp.concatenate([src, jnp.zeros((epad,), jnp.int32)]
                            ).reshape(EPAD // CH, CH)
    dst_g = jnp.concatenate([dst, jnp.zeros((epad,), jnp.int32)]
                            ).reshape(EPAD // CH, CH)
    dst_s = jnp.concatenate([dst, jnp.full((epad,), N, jnp.int32)]
                            ).reshape(EPAD // CH, CH)
    eat = jnp.pad(edge_attr.T, ((0, 12), (0, epad)))
    ea16 = eat.reshape(16, EPK, 8).transpose(1, 2, 0).reshape(EPK, 128)

    npad = NPAD - N
    xs16 = jnp.pad(jnp.concatenate([x, x[:, :2]], axis=1),
                   ((0, npad), (0, 8))).reshape(NPK, 128)
    bat_p = jnp.broadcast_to(
        jnp.pad(batch, (0, npad), constant_values=NG - 1
                ).reshape(NPK, 8)[:, :, None],
        (NPK, 8, 16)).reshape(NPK, 128)

    def conv_stack(b):
        h = lax.conv_general_dilated(b, p['w_c1'], (1,), 'VALID',
                                     dimension_numbers=('NCH', 'OIH', 'NCH'))
        h = h + p['b_c1'][None, :, None]
        h = lax.conv_general_dilated(h, p['w_c2'], (2,), 'VALID',
                                     dimension_numbers=('NCH', 'OIH', 'NCH'))
        h = h + p['b_c2'][None, :, None]
        h = lax.conv_general_dilated(h, p['w_c3'], (2,), 'VALID',
                                     dimension_numbers=('NCH', 'OIH', 'NCH'))
        h = h + p['b_c3'][None, :, None]
        h = lax.conv_general_dilated(h, p['w_c4'], (1,), 'VALID',
                                     dimension_numbers=('NCH', 'OIH', 'NCH'))
        h = h + p['b_c4'][None, :, None]
        return jnp.squeeze(h, axis=1)

    basis = jnp.concatenate([jnp.zeros((1, 1, 128), F32),
                             jnp.eye(128, dtype=F32)[:, None, :]], axis=0)
    eb = conv_stack(basis)
    c_eff = eb[0]
    m_eff = eb[1:] - c_eff[None, :]

    bc2 = bc[:, 0, :]

    fc4 = pl.pallas_call(
        _enc_body,
        out_shape=jax.ShapeDtypeStruct((NG, 8), F32),
    )(bc2, m_eff, c_eff[None, :], p['fc_w1'], p['fc_b1'][None, :],
      p['fc_w2'], p['fc_b2'][None, :], p['fc_w3'], p['fc_b3'][None, :])

    n0 = _rows_call(_n0_body, NPK, RN, (xs16, bat_p, fc4),
                    (128, 128, None), 128)

    g0s = _gather_rows(n0, src_p)
    g0d = _gather_rows(n0, dst_g)
    w1 = p['pe0_w1']
    w1e = jnp.pad(w1[32:36], ((0, 12), (0, 0)))
    e0 = _rows_call(
        _emlp0_body, EPK, RE,
        (g0s, g0d, ea16, w1[0:16], w1[16:32], w1e,
         p['pe0_b1'][None, :], p['pe0_w2'], p['pe0_b2'][None, :]),
        (128, 128, 128, None, None, None, None, None, None), 128)
    p0a, p1a = _scatter_partials(e0, dst_s)
    wn = p['pn0_w1']
    xn0 = _rows_call(
        _nmlp0_body, NPK, RN,
        (n0, p0a, p1a, wn[0:16], wn[16:32], p['pn0_b1'][None, :],
         p['pn0_w2'], p['pn0_b2'][None, :]),
        (128, 128, 128, None, None, None, None, None), 128)

    dg0, dg1 = _deg_partials(dst_s)
    nb0, nb1 = _gather_scatter(xn0, src_p, dst_s)
    xs0 = _rows_call(_smooth_body, NPK, RN, (xn0, nb0, nb1, dg0, dg1),
                     (128, 128, 128, 128, 128), 128)

    g1s = _gather_rows(xs0, src_p)
    g1d = _gather_rows(xs0, dst_g)
    w1 = p['pe1_w1']
    e1 = _rows_call(
        _emlp1_body, EPK, RE,
        (g1s, g1d, g0s, g0d, e0, w1[0:16], w1[16:18], w1[18:34], w1[34:36],
         w1[36:52], p['pe1_b1'][None, :], p['pe1_w2'], p['pe1_b2'][None, :]),
        (128, 128, 128, 128, 128, None, None, None, None, None, None, None,
         None), 128)
    p0b, p1b = _scatter_partials(e1, dst_s)
    wn = p['pn1_w1']
    xn1 = _rows_call(
        _nmlp1_body, NPK, RN,
        (xs0, n0, p0b, p1b, wn[0:16], wn[16:18], wn[18:34],
         p['pn1_b1'][None, :], p['pn1_w2'], p['pn1_b2'][None, :]),
        (128, 128, 128, 128, None, None, None, None, None, None), 128)

    nb0, nb1 = _gather_scatter(xn1, src_p, dst_s)
    pred_p = _rows_call(_smooth_dec_body, NPK, RN,
                        (xn1, nb0, nb1, dg0, dg1, p['dec_w'],
                         p['dec_b'][None, :]),
                        (128, 128, 128, 128, 128, None, None), 32)
    return pred_p.reshape(NPAD, 4)[:N]


# speedup vs baseline: 1.3120x; 1.2910x over previous
import jax
import jax.numpy as jnp
from jax import lax
from jax.experimental import pallas as pl
from jax.experimental.pallas import tpu as pltpu
from jax.experimental.pallas import tpu_sc as plsc

N = 50000
NPAD = 51200
NPK = NPAD // 8
E = 800000
EPAD = 819200
EPK = EPAD // 8
NG = 4
RE = 1024
RN = 800
F32 = jnp.float32

NC = 2
NS = 16
NW = NC * NS
CH = 128
NCH_W = EPAD // (NW * CH)
KG = 8
NGRP = NCH_W // KG
TROWS = NPAD // NS

_SC_PARAMS = pltpu.CompilerParams(use_tc_tiling_on_sc=False)
_MESH_CACHE = []


def _mesh():
    if not _MESH_CACHE:
        _MESH_CACHE.append(
            plsc.VectorSubcoreMesh(core_axis_name="c", subcore_axis_name="s",
                                   num_cores=NC, num_subcores=NS))
    return _MESH_CACHE[0]


def _mm(a, b):
    return jnp.dot(a, b, preferred_element_type=F32)


def _enc_body(bc_ref, m_ref, c_ref, w1_ref, b1_ref, w2_ref, b2_ref,
              w3_ref, b3_ref, out_ref):
    enc = _mm(bc_ref[...], m_ref[...]) + c_ref[...]
    f = jnp.maximum(_mm(enc, w1_ref[...]) + b1_ref[...], 0.0)
    f = jnp.maximum(_mm(f, w2_ref[...]) + b2_ref[...], 0.0)
    out_ref[...] = _mm(f, w3_ref[...]) + b3_ref[...]


def _n0_body(xs_ref, bat_ref, fc4_ref, out_ref):
    xs = xs_ref[...]
    bat = bat_ref[...]
    acc = jnp.zeros(xs.shape, F32)
    zero8 = jnp.zeros((1, 8), F32)
    for g in range(NG):
        row16 = jnp.concatenate([zero8, fc4_ref[g:g + 1, :]], axis=1)
        row = jnp.concatenate([row16] * 8, axis=1)
        acc = acc + jnp.where(bat == g, row, 0.0)
    out_ref[...] = xs + acc


def _emlp0_body(gs_ref, gd_ref, ea_ref, w1s_ref, w1d_ref, w1e_ref, b1_ref,
                w2_ref, b2_ref, out_ref):
    gs = gs_ref[...]
    gd = gd_ref[...]
    ea = ea_ref[...]
    for k in range(8):
        o = 16 * k
        m = _mm(gs[:, o:o + 16], w1s_ref[...])
        m = m + _mm(gd[:, o:o + 16], w1d_ref[...])
        m = m + _mm(ea[:, o:o + 16], w1e_ref[...])
        m = jnp.maximum(m + b1_ref[...], 0.0)
        out_ref[:, o:o + 16] = jnp.maximum(_mm(m, w2_ref[...]) + b2_ref[...],
                                           0.0)


def _emlp1_body(g1s_ref, g1d_ref, g0s_ref, g0d_ref, e0_ref, wa_ref, wb_ref,
                wc_ref, wd_ref, we_ref, b1_ref, w2_ref, b2_ref, out_ref):
    g1s = g1s_ref[...]
    g1d = g1d_ref[...]
    g0s = g0s_ref[...]
    g0d = g0d_ref[...]
    e0 = e0_ref[...]
    for k in range(8):
        o = 16 * k
        a = g1s[:, o:o + 16]
        d = g1d[:, o:o + 16]
        es = (e0[:, o:o + 16] + a + d) * (1.0 / 3.0)
        m = _mm(a, wa_ref[...])
        m = m + _mm(g0s[:, o + 6:o + 8], wb_ref[...])
        m = m + _mm(d, wc_ref[...])
        m = m + _mm(g0d[:, o + 6:o + 8], wd_ref[...])
        m = m + _mm(es, we_ref[...])
        m = jnp.maximum(m + b1_ref[...], 0.0)
        out_ref[:, o:o + 16] = jnp.maximum(_mm(m, w2_ref[...]) + b2_ref[...],
                                           0.0)


def _nmlp0_body(n_ref, p0_ref, p1_ref, wa_ref, wb_ref, b1_ref, w2_ref,
                b2_ref, out_ref):
    n = n_ref[...]
    agg = p0_ref[...] + p1_ref[...]
    for k in range(8):
        o = 16 * k
        h = _mm(n[:, o:o + 16], wa_ref[...]) + _mm(agg[:, o:o + 16],
                                                   wb_ref[...])
        h = jnp.maximum(h + b1_ref[...], 0.0)
        out_ref[:, o:o + 16] = jnp.maximum(_mm(h, w2_ref[...]) + b2_ref[...],
                                           0.0)


def _nmlp1_body(xs_ref, n0_ref, p0_ref, p1_ref, wa_ref, wb_ref, wc_ref,
                b1_ref, w2_ref, b2_ref, out_ref):
    xs = xs_ref[...]
    n0 = n0_ref[...]
    agg = p0_ref[...] + p1_ref[...]
    for k in range(8):
        o = 16 * k
        h = _mm(xs[:, o:o + 16], wa_ref[...])
        h = h + _mm(n0[:, o + 6:o + 8], wb_ref[...])
        h = h + _mm(agg[:, o:o + 16], wc_ref[...])
        h = jnp.maximum(h + b1_ref[...], 0.0)
        out_ref[:, o:o + 16] = jnp.maximum(_mm(h, w2_ref[...]) + b2_ref[...],
                                           0.0)


def _smooth_body(xn_ref, nb0_ref, nb1_ref, dg0_ref, dg1_ref, out_ref):
    xn = xn_ref[...]
    nbr = nb0_ref[...] + nb1_ref[...]
    dg = dg0_ref[...] + dg1_ref[...]
    for k in range(8):
        o = 16 * k
        deg = dg[:, o:o + 1]
        out_ref[:, o:o + 16] = ((xn[:, o:o + 16] + nbr[:, o:o + 16])
                                / (1.0 + deg))


def _smooth_dec_body(xn_ref, nb0_ref, nb1_ref, dg0_ref, dg1_ref, dw_ref,
                     db_ref, out_ref):
    xn = xn_ref[...]
    nbr = nb0_ref[...] + nb1_ref[...]
    dg = dg0_ref[...] + dg1_ref[...]
    for k in range(8):
        o = 16 * k
        deg = dg[:, o:o + 1]
        xs = (xn[:, o:o + 16] + nbr[:, o:o + 16]) / (1.0 + deg)
        out_ref[:, 4 * k:4 * k + 4] = _mm(xs, dw_ref[...]) + db_ref[...]


def _row_spec(b, w):
    return pl.BlockSpec((b, w), lambda i: (i, 0))


def _full_spec(shape):
    return pl.BlockSpec(shape, lambda i: tuple(0 for _ in shape))


def _rows_call(body, nrows, block, in_arrays, widths, out_w):
    grid = nrows // block
    specs = []
    for a, w in zip(in_arrays, widths):
        specs.append(_row_spec(block, w) if w is not None else _full_spec(a.shape))
    return pl.pallas_call(
        body,
        grid=(grid,),
        in_specs=specs,
        out_specs=_row_spec(block, out_w),
        out_shape=jax.ShapeDtypeStruct((nrows, out_w), F32),
    )(*in_arrays)


def _worker_chunk_base():
    wid = lax.axis_index("s") * NC + lax.axis_index("c")
    return wid * NCH_W


def _fill_rows(buf, vec):
    def body(r, _):
        buf[r] = vec
        return 0
    lax.fori_loop(0, CH, body, 0)


def _zero_acc_slice(zbuf, acc):
    t = lax.axis_index("s")
    def body(j, _):
        pltpu.sync_copy(zbuf, acc.at[pl.ds(t * TROWS + j * CH, CH)])
        return 0
    lax.fori_loop(0, TROWS // CH, body, 0)


def _copy_out(acc, p0, p1, obuf):
    c = lax.axis_index("c")
    t = lax.axis_index("s")
    def body(j, _):
        r0 = t * TROWS + j * CH
        pltpu.sync_copy(acc.at[pl.ds(r0, CH)], obuf)
        @pl.when(c == 0)
        def _():
            pltpu.sync_copy(obuf, p0.at[pl.ds(r0, CH)])
        @pl.when(c == 1)
        def _():
            pltpu.sync_copy(obuf, p1.at[pl.ds(r0, CH)])
        return 0
    lax.fori_loop(0, TROWS // CH, body, 0)


def _gather_kernel_body(tab, idx, out, idx_v, rbufs, gsem, wsem):
    base = _worker_chunk_base()
    pltpu.sync_copy(idx.at[pl.ds(base, NCH_W)], idx_v)

    def issue(g, s):
        for b in range(KG):
            pltpu.async_copy(tab.at[idx_v.at[g * KG + b]],
                             rbufs.at[s].at[b], gsem)

    issue(0, 0)

    def body(i, _):
        s = lax.rem(i, 2)
        for b in range(KG):
            pltpu.make_async_copy(tab.at[idx_v.at[i * KG + b]],
                                  rbufs.at[s].at[b], gsem).wait()
        @pl.when(i + 1 < NGRP)
        def _():
            issue(i + 1, 1 - s)
        for b in range(KG):
            pltpu.async_copy(rbufs.at[s].at[b],
                             out.at[pl.ds((base + i * KG + b) * CH, CH)],
                             wsem)
        for b in range(KG):
            pltpu.make_async_copy(rbufs.at[s].at[b],
                                  out.at[pl.ds((base + i * KG + b) * CH, CH)],
                                  wsem).wait()
        return 0

    lax.fori_loop(0, NGRP, body, 0)


def _gather_rows(table, idx2d):
    out = pl.kernel(
        _gather_kernel_body,
        out_type=jax.ShapeDtypeStruct((EPAD, 16), F32),
        mesh=_mesh(),
        compiler_params=_SC_PARAMS,
        scratch_types=[
            pltpu.VMEM((NCH_W, CH), jnp.int32),
            pltpu.VMEM((2, KG, CH, 16), F32),
            pltpu.SemaphoreType.DMA,
            pltpu.SemaphoreType.DMA,
        ],
    )(table.reshape(NPAD, 16), idx2d)
    return out.reshape(EPK, 128)


def _scatter_kernel_body(vals, idx, p0, p1, idx_v, vbufs, sbuf, zbuf,
                         obuf, acc, lsem):
    base = _worker_chunk_base()
    pltpu.sync_copy(idx.at[pl.ds(base, NCH_W)], idx_v)
    _fill_rows(zbuf, jnp.zeros((16,), F32))
    _zero_acc_slice(zbuf, acc)
    plsc.subcore_barrier()

    PR = CH // 8

    def issue(g, s):
        for b in range(KG):
            pltpu.async_copy(vals.at[pl.ds((base + g * KG + b) * PR, PR)],
                             vbufs.at[s].at[b], lsem)

    issue(0, 0)

    def body(i, _):
        s = lax.rem(i, 2)
        for b in range(KG):
            pltpu.make_async_copy(vals.at[pl.ds((base + i * KG + b) * PR, PR)],
                                  vbufs.at[s].at[b], lsem).wait()
        @pl.when(i + 1 < NGRP)
        def _():
            issue(i + 1, 1 - s)
        for b in range(KG):
            pkd = vbufs.at[s].at[b]
            def tr(r, _):
                for k in range(8):
                    sbuf[8 * r + k] = pkd[r, 16 * k:16 * k + 16]
                return 0
            lax.fori_loop(0, PR, tr, 0)
            pltpu.sync_copy(sbuf, acc.at[idx_v.at[i * KG + b]], add=True)
        return 0

    lax.fori_loop(0, NGRP, body, 0)
    plsc.subcore_barrier()
    _copy_out(acc, p0, p1, obuf)


def _scatter_partials(vals, dst2d):
    sds = jax.ShapeDtypeStruct((NPAD, 16), F32)
    p0, p1 = pl.kernel(
        _scatter_kernel_body,
        out_type=(sds, sds),
        mesh=_mesh(),
        compiler_params=_SC_PARAMS,
        scratch_types=[
            pltpu.VMEM((NCH_W, CH), jnp.int32),
            pltpu.VMEM((2, KG, CH // 8, 128), F32),
            pltpu.VMEM((CH, 16), F32),
            pltpu.VMEM((CH, 16), F32),
            pltpu.VMEM((CH, 16), F32),
            pltpu.VMEM_SHARED((NPAD, 16), F32),
            pltpu.SemaphoreType.DMA,
        ],
    )(vals, dst2d)
    return p0.reshape(NPK, 128), p1.reshape(NPK, 128)


def _deg_kernel_body(idx, d0, d1, idx_v, ones_v, zbuf, obuf, acc):
    base = _worker_chunk_base()
    pltpu.sync_copy(idx.at[pl.ds(base, NCH_W)], idx_v)
    _fill_rows(zbuf, jnp.zeros((16,), F32))
    _fill_rows(ones_v, jnp.ones((16,), F32))
    _zero_acc_slice(zbuf, acc)
    plsc.subcore_barrier()

    def body(i, _):
        pltpu.sync_copy(ones_v, acc.at[idx_v.at[i]], add=True)
        return 0

    lax.fori_loop(0, NCH_W, body, 0)
    plsc.subcore_barrier()
    _copy_out(acc, d0, d1, obuf)


def _deg_partials(dst2d):
    sds = jax.ShapeDtypeStruct((NPAD, 16), F32)
    d0, d1 = pl.kernel(
        _deg_kernel_body,
        out_type=(sds, sds),
        mesh=_mesh(),
        compiler_params=_SC_PARAMS,
        scratch_types=[
            pltpu.VMEM((NCH_W, CH), jnp.int32),
            pltpu.VMEM((CH, 16), F32),
            pltpu.VMEM((CH, 16), F32),
            pltpu.VMEM((CH, 16), F32),
            pltpu.VMEM_SHARED((NPAD, 16), F32),
        ],
    )(dst2d)
    return d0.reshape(NPK, 128), d1.reshape(NPK, 128)


def _gs_kernel_body(tab, idx_s, idx_d, p0, p1, sidx_v, didx_v, vbufs, zbuf,
                    obuf, acc, lsem):
    KF = KG // 2
    base = _worker_chunk_base()
    pltpu.sync_copy(idx_s.at[pl.ds(base, NCH_W)], sidx_v)
    pltpu.sync_copy(idx_d.at[pl.ds(base, NCH_W)], didx_v)
    _fill_rows(zbuf, jnp.zeros((16,), F32))
    _zero_acc_slice(zbuf, acc)
    plsc.subcore_barrier()

    def issue(g, s):
        for b in range(KF):
            pltpu.async_copy(tab.at[sidx_v.at[g * KF + b]],
                             vbufs.at[s].at[b], lsem)

    issue(0, 0)
    ngrp = NCH_W // KF

    def body(i, _):
        s = lax.rem(i, 2)
        for b in range(KF):
            pltpu.make_async_copy(tab.at[sidx_v.at[i * KF + b]],
                                  vbufs.at[s].at[b], lsem).wait()
        @pl.when(i + 1 < ngrp)
        def _():
            issue(i + 1, 1 - s)
        for b in range(KF):
            pltpu.sync_copy(vbufs.at[s].at[b],
                            acc.at[didx_v.at[i * KF + b]], add=True)
        return 0

    lax.fori_loop(0, ngrp, body, 0)
    plsc.subcore_barrier()
    _copy_out(acc, p0, p1, obuf)


def _gather_scatter(table, sidx2d, didx2d):
    KF = KG // 2
    sds = jax.ShapeDtypeStruct((NPAD, 16), F32)
    p0, p1 = pl.kernel(
        _gs_kernel_body,
        out_type=(sds, sds),
        mesh=_mesh(),
        compiler_params=_SC_PARAMS,
        scratch_types=[
            pltpu.VMEM((NCH_W, CH), jnp.int32),
            pltpu.VMEM((NCH_W, CH), jnp.int32),
            pltpu.VMEM((2, KF, CH, 16), F32),
            pltpu.VMEM((CH, 16), F32),
            pltpu.VMEM((CH, 16), F32),
            pltpu.VMEM_SHARED((NPAD, 16), F32),
            pltpu.SemaphoreType.DMA,
        ],
    )(table.reshape(NPAD, 16), sidx2d, didx2d)
    return p0.reshape(NPK, 128), p1.reshape(NPK, 128)


def kernel(x, edge_index, edge_attr, bc, batch, params):
    p = params
    src = edge_index[0]
    dst = edge_index[1]
    epad = EPAD - E
    src_p = jnp.concatenate([src, jnp.zeros((epad,), jnp.int32)]
                            ).reshape(EPAD // CH, CH)
    dst_g = jnp.concatenate([dst, jnp.zeros((epad,), jnp.int32)]
                            ).reshape(EPAD // CH, CH)
    dst_s = jnp.concatenate([dst, jnp.full((epad,), N, jnp.int32)]
                            ).reshape(EPAD // CH, CH)
    one = 1.0 + 0.0 * p['dec_b'][0]
    ea16 = (jnp.pad(edge_attr, ((0, epad), (0, 12))) * one).reshape(EPK, 128)

    npad = NPAD - N
    xs16 = jnp.pad(jnp.concatenate([x, x[:, :2]], axis=1),
                   ((0, npad), (0, 8))).reshape(NPK, 128)
    bat_p = jnp.broadcast_to(
        jnp.pad(batch, (0, npad), constant_values=NG - 1
                ).reshape(NPK, 8)[:, :, None],
        (NPK, 8, 16)).reshape(NPK, 128)

    def conv_stack(b):
        h = lax.conv_general_dilated(b, p['w_c1'], (1,), 'VALID',
                                     dimension_numbers=('NCH', 'OIH', 'NCH'))
        h = h + p['b_c1'][None, :, None]
        h = lax.conv_general_dilated(h, p['w_c2'], (2,), 'VALID',
                                     dimension_numbers=('NCH', 'OIH', 'NCH'))
        h = h + p['b_c2'][None, :, None]
        h = lax.conv_general_dilated(h, p['w_c3'], (2,), 'VALID',
                                     dimension_numbers=('NCH', 'OIH', 'NCH'))
        h = h + p['b_c3'][None, :, None]
        h = lax.conv_general_dilated(h, p['w_c4'], (1,), 'VALID',
                                     dimension_numbers=('NCH', 'OIH', 'NCH'))
        h = h + p['b_c4'][None, :, None]
        return jnp.squeeze(h, axis=1)

    basis = jnp.concatenate([jnp.zeros((1, 1, 128), F32),
                             jnp.eye(128, dtype=F32)[:, None, :]], axis=0)
    eb = conv_stack(basis)
    c_eff = eb[0]
    m_eff = eb[1:] - c_eff[None, :]

    bc2 = bc[:, 0, :]

    fc4 = pl.pallas_call(
        _enc_body,
        out_shape=jax.ShapeDtypeStruct((NG, 8), F32),
    )(bc2, m_eff, c_eff[None, :], p['fc_w1'], p['fc_b1'][None, :],
      p['fc_w2'], p['fc_b2'][None, :], p['fc_w3'], p['fc_b3'][None, :])

    n0 = _rows_call(_n0_body, NPK, RN, (xs16, bat_p, fc4),
                    (128, 128, None), 128)

    g0s = _gather_rows(n0, src_p)
    g0d = _gather_rows(n0, dst_g)
    w1 = p['pe0_w1']
    w1e = jnp.pad(w1[32:36], ((0, 12), (0, 0)))
    e0 = _rows_call(
        _emlp0_body, EPK, RE,
        (g0s, g0d, ea16, w1[0:16], w1[16:32], w1e,
         p['pe0_b1'][None, :], p['pe0_w2'], p['pe0_b2'][None, :]),
        (128, 128, 128, None, None, None, None, None, None), 128)
    p0a, p1a = _scatter_partials(e0, dst_s)
    wn = p['pn0_w1']
    xn0 = _rows_call(
        _nmlp0_body, NPK, RN,
        (n0, p0a, p1a, wn[0:16], wn[16:32], p['pn0_b1'][None, :],
         p['pn0_w2'], p['pn0_b2'][None, :]),
        (128, 128, 128, None, None, None, None, None), 128)

    dg0, dg1 = _deg_partials(dst_s)
    nb0, nb1 = _gather_scatter(xn0, src_p, dst_s)
    xs0 = _rows_call(_smooth_body, NPK, RN, (xn0, nb0, nb1, dg0, dg1),
                     (128, 128, 128, 128, 128), 128)

    g1s = _gather_rows(xs0, src_p)
    g1d = _gather_rows(xs0, dst_g)
    w1 = p['pe1_w1']
    e1 = _rows_call(
        _emlp1_body, EPK, RE,
        (g1s, g1d, g0s, g0d, e0, w1[0:16], w1[16:18], w1[18:34], w1[34:36],
         w1[36:52], p['pe1_b1'][None, :], p['pe1_w2'], p['pe1_b2'][None, :]),
        (128, 128, 128, 128, 128, None, None, None, None, None, None, None,
         None), 128)
    p0b, p1b = _scatter_partials(e1, dst_s)
    wn = p['pn1_w1']
    xn1 = _rows_call(
        _nmlp1_body, NPK, RN,
        (xs0, n0, p0b, p1b, wn[0:16], wn[16:18], wn[18:34],
         p['pn1_b1'][None, :], p['pn1_w2'], p['pn1_b2'][None, :]),
        (128, 128, 128, 128, None, None, None, None, None, None), 128)

    nb0, nb1 = _gather_scatter(xn1, src_p, dst_s)
    pred_p = _rows_call(_smooth_dec_body, NPK, RN,
                        (xn1, nb0, nb1, dg0, dg1, p['dec_w'],
                         p['dec_b'][None, :]),
                        (128, 128, 128, 128, 128, None, None), 32)
    return pred_p.reshape(NPAD, 4)[:N]


# speedup vs baseline: 1.3335x; 1.0163x over previous
import jax
import jax.numpy as jnp
from jax import lax
from jax.experimental import pallas as pl
from jax.experimental.pallas import tpu as pltpu
from jax.experimental.pallas import tpu_sc as plsc

N = 50000
NPAD = 51200
NPK = NPAD // 8
E = 800000
EPAD = 819200
EPK = EPAD // 8
NG = 4
RE = 1024
RN = 800
F32 = jnp.float32

NC = 2
NS = 16
NW = NC * NS
CH = 128
NCH_W = EPAD // (NW * CH)
KG = 8
NGRP = NCH_W // KG
TROWS = NPAD // NS

_SC_PARAMS = pltpu.CompilerParams(use_tc_tiling_on_sc=False)
_MESH_CACHE = []


def _mesh():
    if not _MESH_CACHE:
        _MESH_CACHE.append(
            plsc.VectorSubcoreMesh(core_axis_name="c", subcore_axis_name="s",
                                   num_cores=NC, num_subcores=NS))
    return _MESH_CACHE[0]


def _mm(a, b):
    return jnp.dot(a, b, preferred_element_type=F32)


def _enc_body(bc_ref, m_ref, c_ref, w1_ref, b1_ref, w2_ref, b2_ref,
              w3_ref, b3_ref, out_ref):
    enc = _mm(bc_ref[...], m_ref[...]) + c_ref[...]
    f = jnp.maximum(_mm(enc, w1_ref[...]) + b1_ref[...], 0.0)
    f = jnp.maximum(_mm(f, w2_ref[...]) + b2_ref[...], 0.0)
    out_ref[...] = _mm(f, w3_ref[...]) + b3_ref[...]


def _n0_body(xs_ref, bat_ref, fc4_ref, out_ref):
    xs = xs_ref[...]
    bat = bat_ref[...]
    acc = jnp.zeros(xs.shape, F32)
    zero8 = jnp.zeros((1, 8), F32)
    for g in range(NG):
        row16 = jnp.concatenate([zero8, fc4_ref[g:g + 1, :]], axis=1)
        row = jnp.concatenate([row16] * 8, axis=1)
        acc = acc + jnp.where(bat == g, row, 0.0)
    out_ref[...] = xs + acc


def _emlp0_body(gs_ref, gd_ref, ea_ref, w1s_ref, w1d_ref, w1e_ref, b1_ref,
                w2_ref, b2_ref, out_ref):
    gs = gs_ref[...]
    gd = gd_ref[...]
    ea = ea_ref[...]
    for k in range(8):
        o = 16 * k
        m = _mm(gs[:, o:o + 16], w1s_ref[...])
        m = m + _mm(gd[:, o:o + 16], w1d_ref[...])
        m = m + _mm(ea[:, o:o + 16], w1e_ref[...])
        m = jnp.maximum(m + b1_ref[...], 0.0)
        out_ref[:, o:o + 16] = jnp.maximum(_mm(m, w2_ref[...]) + b2_ref[...],
                                           0.0)


def _emlp1_body(g1s_ref, g1d_ref, g0s_ref, g0d_ref, e0_ref, wa_ref, wb_ref,
                wc_ref, wd_ref, we_ref, b1_ref, w2_ref, b2_ref, out_ref):
    g1s = g1s_ref[...]
    g1d = g1d_ref[...]
    g0s = g0s_ref[...]
    g0d = g0d_ref[...]
    e0 = e0_ref[...]
    for k in range(8):
        o = 16 * k
        a = g1s[:, o:o + 16]
        d = g1d[:, o:o + 16]
        es = (e0[:, o:o + 16] + a + d) * (1.0 / 3.0)
        m = _mm(a, wa_ref[...])
        m = m + _mm(g0s[:, o + 6:o + 8], wb_ref[...])
        m = m + _mm(d, wc_ref[...])
        m = m + _mm(g0d[:, o + 6:o + 8], wd_ref[...])
        m = m + _mm(es, we_ref[...])
        m = jnp.maximum(m + b1_ref[...], 0.0)
        out_ref[:, o:o + 16] = jnp.maximum(_mm(m, w2_ref[...]) + b2_ref[...],
                                           0.0)


def _nmlp0_body(n_ref, p0_ref, p1_ref, wa_ref, wb_ref, b1_ref, w2_ref,
                b2_ref, out_ref):
    n = n_ref[...]
    agg = p0_ref[...] + p1_ref[...]
    for k in range(8):
        o = 16 * k
        h = _mm(n[:, o:o + 16], wa_ref[...]) + _mm(agg[:, o:o + 16],
                                                   wb_ref[...])
        h = jnp.maximum(h + b1_ref[...], 0.0)
        out_ref[:, o:o + 16] = jnp.maximum(_mm(h, w2_ref[...]) + b2_ref[...],
                                           0.0)


def _nmlp1_body(xs_ref, n0_ref, p0_ref, p1_ref, wa_ref, wb_ref, wc_ref,
                b1_ref, w2_ref, b2_ref, out_ref):
    xs = xs_ref[...]
    n0 = n0_ref[...]
    agg = p0_ref[...] + p1_ref[...]
    for k in range(8):
        o = 16 * k
        h = _mm(xs[:, o:o + 16], wa_ref[...])
        h = h + _mm(n0[:, o + 6:o + 8], wb_ref[...])
        h = h + _mm(agg[:, o:o + 16], wc_ref[...])
        h = jnp.maximum(h + b1_ref[...], 0.0)
        out_ref[:, o:o + 16] = jnp.maximum(_mm(h, w2_ref[...]) + b2_ref[...],
                                           0.0)


def _smooth_body(xn_ref, nb0_ref, nb1_ref, dg0_ref, dg1_ref, out_ref):
    xn = xn_ref[...]
    nbr = nb0_ref[...] + nb1_ref[...]
    dg = dg0_ref[...] + dg1_ref[...]
    for k in range(8):
        o = 16 * k
        deg = dg[:, o:o + 1]
        out_ref[:, o:o + 16] = ((xn[:, o:o + 16] + nbr[:, o:o + 16])
                                / (1.0 + deg))


def _smooth_dec_body(xn_ref, nb0_ref, nb1_ref, dg0_ref, dg1_ref, dw_ref,
                     db_ref, out_ref):
    xn = xn_ref[...]
    nbr = nb0_ref[...] + nb1_ref[...]
    dg = dg0_ref[...] + dg1_ref[...]
    for k in range(8):
        o = 16 * k
        deg = dg[:, o:o + 1]
        xs = (xn[:, o:o + 16] + nbr[:, o:o + 16]) / (1.0 + deg)
        out_ref[:, 4 * k:4 * k + 4] = _mm(xs, dw_ref[...]) + db_ref[...]


def _row_spec(b, w):
    return pl.BlockSpec((b, w), lambda i: (i, 0))


def _full_spec(shape):
    return pl.BlockSpec(shape, lambda i: tuple(0 for _ in shape))


def _rows_call(body, nrows, block, in_arrays, widths, out_w):
    grid = nrows // block
    specs = []
    for a, w in zip(in_arrays, widths):
        specs.append(_row_spec(block, w) if w is not None else _full_spec(a.shape))
    return pl.pallas_call(
        body,
        grid=(grid,),
        in_specs=specs,
        out_specs=_row_spec(block, out_w),
        out_shape=jax.ShapeDtypeStruct((nrows, out_w), F32),
    )(*in_arrays)


def _worker_chunk_base():
    wid = lax.axis_index("s") * NC + lax.axis_index("c")
    return wid * NCH_W


def _fill_rows(buf, vec):
    def body(r, _):
        buf[r] = vec
        return 0
    lax.fori_loop(0, CH, body, 0)


def _zero_acc_slice(zbuf, acc):
    t = lax.axis_index("s")
    def body(j, _):
        pltpu.sync_copy(zbuf, acc.at[pl.ds(t * TROWS + j * CH, CH)])
        return 0
    lax.fori_loop(0, TROWS // CH, body, 0)


def _copy_out(acc, p0, p1, obuf):
    c = lax.axis_index("c")
    t = lax.axis_index("s")
    def body(j, _):
        r0 = t * TROWS + j * CH
        pltpu.sync_copy(acc.at[pl.ds(r0, CH)], obuf)
        @pl.when(c == 0)
        def _():
            pltpu.sync_copy(obuf, p0.at[pl.ds(r0, CH)])
        @pl.when(c == 1)
        def _():
            pltpu.sync_copy(obuf, p1.at[pl.ds(r0, CH)])
        return 0
    lax.fori_loop(0, TROWS // CH, body, 0)


def _gather_kernel_body(tab, idx, out, idx_v, rbufs, gsem, wsem):
    base = _worker_chunk_base()
    pltpu.sync_copy(idx.at[pl.ds(base, NCH_W)], idx_v)

    def issue(g, s):
        for b in range(KG):
            pltpu.async_copy(tab.at[idx_v.at[g * KG + b]],
                             rbufs.at[s].at[b], gsem)

    issue(0, 0)

    def body(i, _):
        s = lax.rem(i, 2)
        for b in range(KG):
            pltpu.make_async_copy(tab.at[idx_v.at[i * KG + b]],
                                  rbufs.at[s].at[b], gsem).wait()
        @pl.when(i + 1 < NGRP)
        def _():
            issue(i + 1, 1 - s)
        for b in range(KG):
            pltpu.async_copy(rbufs.at[s].at[b],
                             out.at[pl.ds((base + i * KG + b) * CH, CH)],
                             wsem)
        for b in range(KG):
            pltpu.make_async_copy(rbufs.at[s].at[b],
                                  out.at[pl.ds((base + i * KG + b) * CH, CH)],
                                  wsem).wait()
        return 0

    lax.fori_loop(0, NGRP, body, 0)


def _gather_rows(table, idx2d):
    out = pl.kernel(
        _gather_kernel_body,
        out_type=jax.ShapeDtypeStruct((EPAD, 16), F32),
        mesh=_mesh(),
        compiler_params=_SC_PARAMS,
        scratch_types=[
            pltpu.VMEM((NCH_W, CH), jnp.int32),
            pltpu.VMEM((2, KG, CH, 16), F32),
            pltpu.SemaphoreType.DMA,
            pltpu.SemaphoreType.DMA,
        ],
    )(table.reshape(NPAD, 16), idx2d)
    return out.reshape(EPK, 128)


def _scatter_kernel_body(vals, idx, p0, p1, idx_v, vbufs, sbufs, zbuf,
                         obuf, acc, lsem, ssem):
    base = _worker_chunk_base()
    pltpu.sync_copy(idx.at[pl.ds(base, NCH_W)], idx_v)
    _fill_rows(zbuf, jnp.zeros((16,), F32))
    _zero_acc_slice(zbuf, acc)
    plsc.subcore_barrier()

    PR = CH // 8

    def issue(g, s):
        for b in range(KG):
            pltpu.async_copy(vals.at[pl.ds((base + g * KG + b) * PR, PR)],
                             vbufs.at[s].at[b], lsem)

    issue(0, 0)

    def body(i, _):
        s = lax.rem(i, 2)
        for b in range(KG):
            pltpu.make_async_copy(vals.at[pl.ds((base + i * KG + b) * PR, PR)],
                                  vbufs.at[s].at[b], lsem).wait()
        @pl.when(i + 1 < NGRP)
        def _():
            issue(i + 1, 1 - s)
        for b in range(KG):
            pkd = vbufs.at[s].at[b]
            sb = sbufs.at[b]
            def tr(r, _):
                for k in range(8):
                    sb[8 * r + k] = pkd[r, 16 * k:16 * k + 16]
                return 0
            lax.fori_loop(0, PR, tr, 0)
            pltpu.async_copy(sb, acc.at[idx_v.at[i * KG + b]], ssem,
                             add=True)
        for b in range(KG):
            pltpu.make_async_copy(sbufs.at[b],
                                  acc.at[idx_v.at[i * KG + b]], ssem).wait()
        return 0

    lax.fori_loop(0, NGRP, body, 0)
    plsc.subcore_barrier()
    _copy_out(acc, p0, p1, obuf)


def _scatter_partials(vals, dst2d):
    sds = jax.ShapeDtypeStruct((NPAD, 16), F32)
    p0, p1 = pl.kernel(
        _scatter_kernel_body,
        out_type=(sds, sds),
        mesh=_mesh(),
        compiler_params=_SC_PARAMS,
        scratch_types=[
            pltpu.VMEM((NCH_W, CH), jnp.int32),
            pltpu.VMEM((2, KG, CH // 8, 128), F32),
            pltpu.VMEM((KG, CH, 16), F32),
            pltpu.VMEM((CH, 16), F32),
            pltpu.VMEM((CH, 16), F32),
            pltpu.VMEM_SHARED((NPAD, 16), F32),
            pltpu.SemaphoreType.DMA,
            pltpu.SemaphoreType.DMA,
        ],
    )(vals, dst2d)
    return p0.reshape(NPK, 128), p1.reshape(NPK, 128)


def _deg_kernel_body(idx, d0, d1, idx_v, ones_v, zbuf, obuf, acc, dsem):
    base = _worker_chunk_base()
    pltpu.sync_copy(idx.at[pl.ds(base, NCH_W)], idx_v)
    _fill_rows(zbuf, jnp.zeros((16,), F32))
    _fill_rows(ones_v, jnp.ones((16,), F32))
    _zero_acc_slice(zbuf, acc)
    plsc.subcore_barrier()

    def body(g, _):
        for b in range(KG):
            pltpu.async_copy(ones_v, acc.at[idx_v.at[g * KG + b]], dsem,
                             add=True)
        for b in range(KG):
            pltpu.make_async_copy(ones_v, acc.at[idx_v.at[g * KG + b]],
                                  dsem).wait()
        return 0

    lax.fori_loop(0, NCH_W // KG, body, 0)
    plsc.subcore_barrier()
    _copy_out(acc, d0, d1, obuf)


def _deg_partials(dst2d):
    sds = jax.ShapeDtypeStruct((NPAD, 16), F32)
    d0, d1 = pl.kernel(
        _deg_kernel_body,
        out_type=(sds, sds),
        mesh=_mesh(),
        compiler_params=_SC_PARAMS,
        scratch_types=[
            pltpu.VMEM((NCH_W, CH), jnp.int32),
            pltpu.VMEM((CH, 16), F32),
            pltpu.VMEM((CH, 16), F32),
            pltpu.VMEM((CH, 16), F32),
            pltpu.VMEM_SHARED((NPAD, 16), F32),
            pltpu.SemaphoreType.DMA,
        ],
    )(dst2d)
    return d0.reshape(NPK, 128), d1.reshape(NPK, 128)


def _gs_kernel_body(tab, idx_s, idx_d, p0, p1, sidx_v, didx_v, vbufs, zbuf,
                    obuf, acc, lsem, ssem):
    KF = KG // 2
    base = _worker_chunk_base()
    pltpu.sync_copy(idx_s.at[pl.ds(base, NCH_W)], sidx_v)
    pltpu.sync_copy(idx_d.at[pl.ds(base, NCH_W)], didx_v)
    _fill_rows(zbuf, jnp.zeros((16,), F32))
    _zero_acc_slice(zbuf, acc)
    plsc.subcore_barrier()

    def issue(g, s):
        for b in range(KF):
            pltpu.async_copy(tab.at[sidx_v.at[g * KF + b]],
                             vbufs.at[s].at[b], lsem)

    issue(0, 0)
    ngrp = NCH_W // KF

    def body(i, _):
        s = lax.rem(i, 2)
        for b in range(KF):
            pltpu.make_async_copy(tab.at[sidx_v.at[i * KF + b]],
                                  vbufs.at[s].at[b], lsem).wait()
        @pl.when(i + 1 < ngrp)
        def _():
            issue(i + 1, 1 - s)
        for b in range(KF):
            pltpu.async_copy(vbufs.at[s].at[b],
                             acc.at[didx_v.at[i * KF + b]], ssem, add=True)
        for b in range(KF):
            pltpu.make_async_copy(vbufs.at[s].at[b],
                                  acc.at[didx_v.at[i * KF + b]], ssem).wait()
        return 0

    lax.fori_loop(0, ngrp, body, 0)
    plsc.subcore_barrier()
    _copy_out(acc, p0, p1, obuf)


def _gather_scatter(table, sidx2d, didx2d):
    KF = KG // 2
    sds = jax.ShapeDtypeStruct((NPAD, 16), F32)
    p0, p1 = pl.kernel(
        _gs_kernel_body,
        out_type=(sds, sds),
        mesh=_mesh(),
        compiler_params=_SC_PARAMS,
        scratch_types=[
            pltpu.VMEM((NCH_W, CH), jnp.int32),
            pltpu.VMEM((NCH_W, CH), jnp.int32),
            pltpu.VMEM((2, KF, CH, 16), F32),
            pltpu.VMEM((CH, 16), F32),
            pltpu.VMEM((CH, 16), F32),
            pltpu.VMEM_SHARED((NPAD, 16), F32),
            pltpu.SemaphoreType.DMA,
            pltpu.SemaphoreType.DMA,
        ],
    )(table.reshape(NPAD, 16), sidx2d, didx2d)
    return p0.reshape(NPK, 128), p1.reshape(NPK, 128)


def kernel(x, edge_index, edge_attr, bc, batch, params):
    p = params
    src = edge_index[0]
    dst = edge_index[1]
    epad = EPAD - E
    src_p = jnp.concatenate([src, jnp.zeros((epad,), jnp.int32)]
                            ).reshape(EPAD // CH, CH)
    dst_g = jnp.concatenate([dst, jnp.zeros((epad,), jnp.int32)]
                            ).reshape(EPAD // CH, CH)
    dst_s = jnp.concatenate([dst, jnp.full((epad,), N, jnp.int32)]
                            ).reshape(EPAD // CH, CH)
    one = 1.0 + 0.0 * p['dec_b'][0]
    ea16 = (jnp.pad(edge_attr, ((0, epad), (0, 12))) * one).reshape(EPK, 128)

    npad = NPAD - N
    xs16 = jnp.pad(jnp.concatenate([x, x[:, :2]], axis=1),
                   ((0, npad), (0, 8))).reshape(NPK, 128)
    bat_p = jnp.broadcast_to(
        jnp.pad(batch, (0, npad), constant_values=NG - 1
                ).reshape(NPK, 8)[:, :, None],
        (NPK, 8, 16)).reshape(NPK, 128)

    def conv_stack(b):
        h = lax.conv_general_dilated(b, p['w_c1'], (1,), 'VALID',
                                     dimension_numbers=('NCH', 'OIH', 'NCH'))
        h = h + p['b_c1'][None, :, None]
        h = lax.conv_general_dilated(h, p['w_c2'], (2,), 'VALID',
                                     dimension_numbers=('NCH', 'OIH', 'NCH'))
        h = h + p['b_c2'][None, :, None]
        h = lax.conv_general_dilated(h, p['w_c3'], (2,), 'VALID',
                                     dimension_numbers=('NCH', 'OIH', 'NCH'))
        h = h + p['b_c3'][None, :, None]
        h = lax.conv_general_dilated(h, p['w_c4'], (1,), 'VALID',
                                     dimension_numbers=('NCH', 'OIH', 'NCH'))
        h = h + p['b_c4'][None, :, None]
        return jnp.squeeze(h, axis=1)

    basis = jnp.concatenate([jnp.zeros((1, 1, 128), F32),
                             jnp.eye(128, dtype=F32)[:, None, :]], axis=0)
    eb = conv_stack(basis)
    c_eff = eb[0]
    m_eff = eb[1:] - c_eff[None, :]

    bc2 = bc[:, 0, :]

    fc4 = pl.pallas_call(
        _enc_body,
        out_shape=jax.ShapeDtypeStruct((NG, 8), F32),
    )(bc2, m_eff, c_eff[None, :], p['fc_w1'], p['fc_b1'][None, :],
      p['fc_w2'], p['fc_b2'][None, :], p['fc_w3'], p['fc_b3'][None, :])

    n0 = _rows_call(_n0_body, NPK, RN, (xs16, bat_p, fc4),
                    (128, 128, None), 128)

    g0s = _gather_rows(n0, src_p)
    g0d = _gather_rows(n0, dst_g)
    w1 = p['pe0_w1']
    w1e = jnp.pad(w1[32:36], ((0, 12), (0, 0)))
    e0 = _rows_call(
        _emlp0_body, EPK, RE,
        (g0s, g0d, ea16, w1[0:16], w1[16:32], w1e,
         p['pe0_b1'][None, :], p['pe0_w2'], p['pe0_b2'][None, :]),
        (128, 128, 128, None, None, None, None, None, None), 128)
    p0a, p1a = _scatter_partials(e0, dst_s)
    wn = p['pn0_w1']
    xn0 = _rows_call(
        _nmlp0_body, NPK, RN,
        (n0, p0a, p1a, wn[0:16], wn[16:32], p['pn0_b1'][None, :],
         p['pn0_w2'], p['pn0_b2'][None, :]),
        (128, 128, 128, None, None, None, None, None), 128)

    dg0, dg1 = _deg_partials(dst_s)
    nb0, nb1 = _gather_scatter(xn0, src_p, dst_s)
    xs0 = _rows_call(_smooth_body, NPK, RN, (xn0, nb0, nb1, dg0, dg1),
                     (128, 128, 128, 128, 128), 128)

    g1s = _gather_rows(xs0, src_p)
    g1d = _gather_rows(xs0, dst_g)
    w1 = p['pe1_w1']
    e1 = _rows_call(
        _emlp1_body, EPK, RE,
        (g1s, g1d, g0s, g0d, e0, w1[0:16], w1[16:18], w1[18:34], w1[34:36],
         w1[36:52], p['pe1_b1'][None, :], p['pe1_w2'], p['pe1_b2'][None, :]),
        (128, 128, 128, 128, 128, None, None, None, None, None, None, None,
         None), 128)
    p0b, p1b = _scatter_partials(e1, dst_s)
    wn = p['pn1_w1']
    xn1 = _rows_call(
        _nmlp1_body, NPK, RN,
        (xs0, n0, p0b, p1b, wn[0:16], wn[16:18], wn[18:34],
         p['pn1_b1'][None, :], p['pn1_w2'], p['pn1_b2'][None, :]),
        (128, 128, 128, 128, None, None, None, None, None, None), 128)

    nb0, nb1 = _gather_scatter(xn1, src_p, dst_s)
    pred_p = _rows_call(_smooth_dec_body, NPK, RN,
                        (xn1, nb0, nb1, dg0, dg1, p['dec_w'],
                         p['dec_b'][None, :]),
                        (128, 128, 128, 128, 128, None, None), 32)
    return pred_p.reshape(NPAD, 4)[:N]
